# Initial kernel scaffold; baseline (speedup 1.0000x reference)
import jax
import jax.numpy as jnp
from jax.experimental import pallas as pl


def kernel(text_input_ids, text_position_ids, text_segment_ids, image_feat, image_position_ids, image_segment_ids, word_emb, pos_emb, type_emb, ln_g, ln_b, img_W, img_b, imgln_g, imgln_b, imgln2_g, imgln2_b):
    raise NotImplementedError("write your pallas kernel here")


# speedup vs baseline: 2.6543x; 2.6543x over previous
import functools

import jax
import jax.numpy as jnp
from jax import lax
from jax.experimental import pallas as pl
from jax.experimental.pallas import tpu as pltpu
from jax.experimental.pallas import tpu_sc as plsc

B, LT, LI = 64, 512, 100
VOCAB, MAXPOS, NTYPES, HIDDEN, IMG_DIM = 30522, 512, 2, 768, 2048
EPS = 1e-12

_NC, _NS = 2, 16
_NW = _NC * _NS
_TOK = B * LT
_PER_W = _TOK // _NW
_CH = 128


def _sc_word_gather(table, idx):
    mesh = plsc.VectorSubcoreMesh(core_axis_name="c", subcore_axis_name="s")

    @functools.partial(
        pl.kernel, mesh=mesh,
        out_type=jax.ShapeDtypeStruct((_TOK, HIDDEN), jnp.float32),
        scratch_types=[
            pltpu.VMEM((_CH,), jnp.int32),
            pltpu.VMEM((_CH, HIDDEN), jnp.float32),
            pltpu.SemaphoreType.DMA,
        ],
    )
    def k(table_hbm, idx_hbm, out_hbm, idx_v, rows_v, sem):
        wid = lax.axis_index("s") * _NC + lax.axis_index("c")

        def body(i, carry):
            base = wid * _PER_W + i * _CH
            pltpu.sync_copy(idx_hbm.at[pl.ds(base, _CH)], idx_v)
            pltpu.async_copy(table_hbm.at[idx_v], rows_v, sem).wait()
            pltpu.sync_copy(rows_v, out_hbm.at[pl.ds(base, _CH)])
            return carry

        lax.fori_loop(0, _PER_W // _CH, body, 0)

    return k(table, idx)


def _ln(x, g, b):
    mu = jnp.mean(x, axis=-1, keepdims=True)
    var = jnp.mean((x - mu) ** 2, axis=-1, keepdims=True)
    return (x - mu) * lax.rsqrt(var + EPS) * g + b


def _tc_body(words_ref, tpos_ref, tseg_ref, feat_ref, ipos_ref, iseg_ref,
             pos_ref, type_ref, w_ref, prm_ref, out_ref):
    ln_g = prm_ref[0, :]
    ln_b = prm_ref[1, :]
    img_b = prm_ref[2, :]
    imgln_g = prm_ref[3, :]
    imgln_b = prm_ref[4, :]
    imgln2_g = prm_ref[5, :]
    imgln2_b = prm_ref[6, :]
    t0 = type_ref[0:1, :]
    t1 = type_ref[1:2, :]
    pos_tab = pos_ref[...]

    tpos = tpos_ref[0]
    oh_t = (tpos == lax.broadcasted_iota(jnp.int32, (LT, MAXPOS), 1)
            ).astype(jnp.float32)
    posrows = jnp.dot(oh_t, pos_tab, preferred_element_type=jnp.float32)
    segrows = jnp.where(tseg_ref[0] == 0, t0, t1)
    txt = _ln(words_ref[0] + posrows + segrows, ln_g, ln_b)

    img = jnp.dot(feat_ref[0], w_ref[...],
                  preferred_element_type=jnp.float32) + img_b
    img = _ln(img, imgln_g, imgln_b)
    oh_i = (ipos_ref[0] == lax.broadcasted_iota(jnp.int32, (LI, MAXPOS), 1)
            ).astype(jnp.float32)
    img = img + jnp.dot(oh_i, pos_tab, preferred_element_type=jnp.float32)
    img = img + jnp.where(iseg_ref[0] == 0, t0, t1)
    img = _ln(img, imgln2_g, imgln2_b)

    out_ref[0, :LT, :] = txt
    out_ref[0, LT:, :] = img


def kernel(text_input_ids, text_position_ids, text_segment_ids, image_feat,
           image_position_ids, image_segment_ids, word_emb, pos_emb, type_emb,
           ln_g, ln_b, img_W, img_b, imgln_g, imgln_b, imgln2_g, imgln2_b):
    wid_flat = text_input_ids.reshape(-1).astype(jnp.int32)
    words = _sc_word_gather(word_emb, wid_flat).reshape(B, LT, HIDDEN)

    tpos = text_position_ids.reshape(B, LT, 1).astype(jnp.int32)
    tseg = text_segment_ids.reshape(B, LT, 1).astype(jnp.int32)
    ipos = image_position_ids.reshape(B, LI, 1).astype(jnp.int32)
    iseg = image_segment_ids.reshape(B, LI, 1).astype(jnp.int32)
    type_pad = jnp.concatenate(
        [type_emb, jnp.zeros((8 - NTYPES, HIDDEN), jnp.float32)], axis=0)
    prm = jnp.stack(
        [ln_g, ln_b, img_b, imgln_g, imgln_b, imgln2_g, imgln2_b,
         jnp.zeros((HIDDEN,), jnp.float32)], axis=0)

    out = pl.pallas_call(
        _tc_body,
        grid=(B,),
        in_specs=[
            pl.BlockSpec((1, LT, HIDDEN), lambda b: (b, 0, 0)),
            pl.BlockSpec((1, LT, 1), lambda b: (b, 0, 0)),
            pl.BlockSpec((1, LT, 1), lambda b: (b, 0, 0)),
            pl.BlockSpec((1, LI, IMG_DIM), lambda b: (b, 0, 0)),
            pl.BlockSpec((1, LI, 1), lambda b: (b, 0, 0)),
            pl.BlockSpec((1, LI, 1), lambda b: (b, 0, 0)),
            pl.BlockSpec((MAXPOS, HIDDEN), lambda b: (0, 0)),
            pl.BlockSpec((8, HIDDEN), lambda b: (0, 0)),
            pl.BlockSpec((IMG_DIM, HIDDEN), lambda b: (0, 0)),
            pl.BlockSpec((8, HIDDEN), lambda b: (0, 0)),
        ],
        out_specs=pl.BlockSpec((1, LT + LI, HIDDEN), lambda b: (b, 0, 0)),
        out_shape=jax.ShapeDtypeStruct((B, LT + LI, HIDDEN), jnp.float32),
        compiler_params=pltpu.CompilerParams(
            dimension_semantics=("arbitrary",)),
    )(words, tpos, tseg, image_feat, ipos, iseg,
      pos_emb, type_pad, img_W, prm)
    return out


# speedup vs baseline: 2.6779x; 1.0089x over previous
import functools

import jax
import jax.numpy as jnp
from jax import lax
from jax.experimental import pallas as pl
from jax.experimental.pallas import tpu as pltpu
from jax.experimental.pallas import tpu_sc as plsc

B, LT, LI = 64, 512, 100
VOCAB, MAXPOS, NTYPES, HIDDEN, IMG_DIM = 30522, 512, 2, 768, 2048
EPS = 1e-12

_NC, _NS = 2, 16
_NW = _NC * _NS
_TOK = B * LT
_PER_W = _TOK // _NW
_CH = 64
_NCH = _PER_W // _CH


def _sc_word_gather(table, idx):
    mesh = plsc.VectorSubcoreMesh(core_axis_name="c", subcore_axis_name="s")

    @functools.partial(
        pl.kernel, mesh=mesh,
        out_type=jax.ShapeDtypeStruct((_TOK, HIDDEN), jnp.float32),
        scratch_types=[
            pltpu.VMEM((_PER_W,), jnp.int32),
            pltpu.VMEM((2, _CH, HIDDEN), jnp.float32),
            pltpu.SemaphoreType.DMA((2,)),
            pltpu.SemaphoreType.DMA((2,)),
        ],
    )
    def k(table_hbm, idx_hbm, out_hbm, idx_v, rows_v, gsem, wsem):
        wid = lax.axis_index("s") * _NC + lax.axis_index("c")
        base = wid * _PER_W
        pltpu.sync_copy(idx_hbm.at[pl.ds(base, _PER_W)], idx_v)

        def g_args(c, b):
            return (table_hbm.at[idx_v.at[pl.ds(c * _CH, _CH)]],
                    rows_v.at[b], gsem.at[b])

        def w_args(c, b):
            return (rows_v.at[b], out_hbm.at[pl.ds(base + c * _CH, _CH)],
                    wsem.at[b])

        pltpu.async_copy(*g_args(0, 0))
        pltpu.async_copy(*g_args(1, 1))

        def body(j, carry):
            for b in range(2):
                c = 2 * j + b
                pltpu.make_async_copy(*g_args(c, b)).wait()
                pltpu.async_copy(*w_args(c, b))

            @pl.when(j < _NCH // 2 - 1)
            def _():
                for b in range(2):
                    c = 2 * j + b
                    pltpu.make_async_copy(*w_args(c, b)).wait()
                    pltpu.async_copy(*g_args(c + 2, b))

            return carry

        lax.fori_loop(0, _NCH // 2, body, 0)
        for b in range(2):
            pltpu.make_async_copy(*w_args(_NCH - 2 + b, b)).wait()

    return k(table, idx)


def _ln(x, g, b):
    mu = jnp.mean(x, axis=-1, keepdims=True)
    var = jnp.mean((x - mu) ** 2, axis=-1, keepdims=True)
    return (x - mu) * lax.rsqrt(var + EPS) * g + b


def _tc_body(words_ref, tpos_ref, tseg_ref, feat_ref, ipos_ref, iseg_ref,
             pos_ref, type_ref, w_ref, prm_ref, out_ref):
    ln_g = prm_ref[0, :]
    ln_b = prm_ref[1, :]
    img_b = prm_ref[2, :]
    imgln_g = prm_ref[3, :]
    imgln_b = prm_ref[4, :]
    imgln2_g = prm_ref[5, :]
    imgln2_b = prm_ref[6, :]
    t0 = type_ref[0:1, :]
    t1 = type_ref[1:2, :]
    pos_tab = pos_ref[...]

    tpos = tpos_ref[0]
    oh_t = (tpos == lax.broadcasted_iota(jnp.int32, (LT, MAXPOS), 1)
            ).astype(jnp.float32)
    posrows = jnp.dot(oh_t, pos_tab, preferred_element_type=jnp.float32)
    segrows = jnp.where(tseg_ref[0] == 0, t0, t1)
    txt = _ln(words_ref[0] + posrows + segrows, ln_g, ln_b)

    img = jnp.dot(feat_ref[0], w_ref[...],
                  preferred_element_type=jnp.float32) + img_b
    img = _ln(img, imgln_g, imgln_b)
    oh_i = (ipos_ref[0] == lax.broadcasted_iota(jnp.int32, (LI, MAXPOS), 1)
            ).astype(jnp.float32)
    img = img + jnp.dot(oh_i, pos_tab, preferred_element_type=jnp.float32)
    img = img + jnp.where(iseg_ref[0] == 0, t0, t1)
    img = _ln(img, imgln2_g, imgln2_b)

    out_ref[0, :LT, :] = txt
    out_ref[0, LT:, :] = img


def kernel(text_input_ids, text_position_ids, text_segment_ids, image_feat,
           image_position_ids, image_segment_ids, word_emb, pos_emb, type_emb,
           ln_g, ln_b, img_W, img_b, imgln_g, imgln_b, imgln2_g, imgln2_b):
    wid_flat = text_input_ids.reshape(-1).astype(jnp.int32)
    words = _sc_word_gather(word_emb, wid_flat).reshape(B, LT, HIDDEN)

    tpos = text_position_ids.reshape(B, LT, 1).astype(jnp.int32)
    tseg = text_segment_ids.reshape(B, LT, 1).astype(jnp.int32)
    ipos = image_position_ids.reshape(B, LI, 1).astype(jnp.int32)
    iseg = image_segment_ids.reshape(B, LI, 1).astype(jnp.int32)
    type_pad = jnp.concatenate(
        [type_emb, jnp.zeros((8 - NTYPES, HIDDEN), jnp.float32)], axis=0)
    prm = jnp.stack(
        [ln_g, ln_b, img_b, imgln_g, imgln_b, imgln2_g, imgln2_b,
         jnp.zeros((HIDDEN,), jnp.float32)], axis=0)

    out = pl.pallas_call(
        _tc_body,
        grid=(B,),
        in_specs=[
            pl.BlockSpec((1, LT, HIDDEN), lambda b: (b, 0, 0)),
            pl.BlockSpec((1, LT, 1), lambda b: (b, 0, 0)),
            pl.BlockSpec((1, LT, 1), lambda b: (b, 0, 0)),
            pl.BlockSpec((1, LI, IMG_DIM), lambda b: (b, 0, 0)),
            pl.BlockSpec((1, LI, 1), lambda b: (b, 0, 0)),
            pl.BlockSpec((1, LI, 1), lambda b: (b, 0, 0)),
            pl.BlockSpec((MAXPOS, HIDDEN), lambda b: (0, 0)),
            pl.BlockSpec((8, HIDDEN), lambda b: (0, 0)),
            pl.BlockSpec((IMG_DIM, HIDDEN), lambda b: (0, 0)),
            pl.BlockSpec((8, HIDDEN), lambda b: (0, 0)),
        ],
        out_specs=pl.BlockSpec((1, LT + LI, HIDDEN), lambda b: (b, 0, 0)),
        out_shape=jax.ShapeDtypeStruct((B, LT + LI, HIDDEN), jnp.float32),
        compiler_params=pltpu.CompilerParams(
            dimension_semantics=("arbitrary",)),
    )(words, tpos, tseg, image_feat, ipos, iseg,
      pos_emb, type_pad, img_W, prm)
    return out
